# baseline (speedup 1.0000x reference)
import functools

import jax
import jax.numpy as jnp
from jax import lax
from jax.experimental import pallas as pl
from jax.experimental.pallas import tpu as pltpu
from jax.experimental.pallas import tpu_sc as plsc

N_NODES = 50000
N_EDGES = 800000
N_GRAPHS = 512
EMB = 64

_NODE_BLK = 2000
_EPAIRS = N_EDGES // 2
_EDGE_BLK2 = 3200

_NC = 2
_NS = 16
_NW = _NC * _NS
_IW = 125
_EROWS = N_EDGES // _IW
_HROWS = _EROWS // 2
_NROWS = N_NODES // _IW

_SC_PARAMS = pltpu.CompilerParams(use_tc_tiling_on_sc=False)


def _ln_relu(z, g, b):
    mu = jnp.mean(z, axis=-1, keepdims=True)
    var = jnp.var(z, axis=-1, keepdims=True)
    return jax.nn.relu((z - mu) * jax.lax.rsqrt(var + 1e-5) * g + b)


def _dot(a, w):
    return jax.lax.dot_general(a, w, (((1,), (0,)), ((), ())),
                               preferred_element_type=jnp.float32)


def _full(shape):
    return pl.BlockSpec(shape, lambda i: (0,) * len(shape))


def _sc_gather_add(u, v, s2d, d2d):
    pairs = _IW
    rows_per_w = _HROWS // _NW
    mrows = 20
    n_megas = rows_per_w // mrows
    n_chunks = mrows
    mesh = plsc.VectorSubcoreMesh(core_axis_name="c", subcore_axis_name="s")

    @functools.partial(
        pl.kernel,
        out_type=jax.ShapeDtypeStruct((_EPAIRS, 2 * EMB), jnp.bfloat16),
        mesh=mesh,
        scratch_types=[
            [pltpu.VMEM((mrows, _IW), jnp.int32)] * 4,
            [pltpu.VMEM((_IW, EMB), jnp.bfloat16)] * 8,
            [pltpu.VMEM((pairs, 2 * EMB), jnp.bfloat16)] * 2,
            [pltpu.SemaphoreType.DMA] * 2,
            [pltpu.SemaphoreType.DMA] * 2,
        ],
        compiler_params=_SC_PARAMS,
    )
    def k(u_hbm, v_hbm, s_hbm, d_hbm, g2_hbm, IDX, AB, O, sg, so):
        c = lax.axis_index("c")
        s = lax.axis_index("s")
        wid = s * _NC + c
        row_base = wid * rows_per_w
        si, di, si2, di2 = IDX

        def fire(l, p):
            pltpu.async_copy(u_hbm.at[si.at[l]], AB[4 * p + 0], sg[p])
            pltpu.async_copy(v_hbm.at[di.at[l]], AB[4 * p + 1], sg[p])
            pltpu.async_copy(u_hbm.at[si2.at[l]], AB[4 * p + 2], sg[p])
            pltpu.async_copy(v_hbm.at[di2.at[l]], AB[4 * p + 3], sg[p])

        def drain_gather(p):
            for r in range(4):
                pltpu.make_async_copy(u_hbm.at[si.at[0]], AB[4 * p + r],
                                      sg[p]).wait()

        def mega(mi, carry):
            r0 = row_base + mi * mrows
            pbase = r0 * _IW
            pltpu.sync_copy(s_hbm.at[pl.ds(r0, mrows)], si)
            pltpu.sync_copy(d_hbm.at[pl.ds(r0, mrows)], di)
            pltpu.sync_copy(s_hbm.at[pl.ds(_HROWS + r0, mrows)], si2)
            pltpu.sync_copy(d_hbm.at[pl.ds(_HROWS + r0, mrows)], di2)
            fire(0, 0)

            def pair(l2, carry2):
                for p in range(2):
                    l = 2 * l2 + p

                    @pl.when(l < n_chunks - 1)
                    def _fire_next():
                        fire(l + 1, 1 - p)

                    drain_gather(p)

                    @pl.when(l >= 2)
                    def _wait_out():
                        pltpu.make_async_copy(
                            O[p], g2_hbm.at[pl.ds(pbase + (l - 2) * pairs,
                                                  pairs)], so[p]).wait()

                    def addrow(m, c3, p=p):
                        for q in range(EMB // 32):
                            O[p][m, pl.ds(q * 32, 32)] = (
                                AB[4 * p + 0][m, pl.ds(q * 32, 32)]
                                + AB[4 * p + 1][m, pl.ds(q * 32, 32)])
                            O[p][m, pl.ds(EMB + q * 32, 32)] = (
                                AB[4 * p + 2][m, pl.ds(q * 32, 32)]
                                + AB[4 * p + 3][m, pl.ds(q * 32, 32)])
                        return c3

                    lax.fori_loop(0, _IW, addrow, 0)
                    pltpu.async_copy(
                        O[p], g2_hbm.at[pl.ds(pbase + l * pairs, pairs)],
                        so[p])
                return carry2

            lax.fori_loop(0, n_chunks // 2, pair, 0)
            for p in range(2):
                pltpu.make_async_copy(
                    O[p], g2_hbm.at[pl.ds(pbase + (n_chunks - 2 + p) * pairs,
                                          pairs)], so[p]).wait()
            return carry

        lax.fori_loop(0, n_megas, mega, 0)

    return k(u, v, s2d, d2d)


def _sc_segment_sum(e2, d2d):
    G = 10
    chunk_e = G * _IW
    pairs = chunk_e
    rows_per_t = _HROWS // _NS
    n_chunks = rows_per_t // G
    stripe = N_NODES // _NS
    zrows = 625
    mesh = plsc.VectorSubcoreMesh(core_axis_name="c", subcore_axis_name="s")

    @functools.partial(
        pl.kernel,
        out_type=jax.ShapeDtypeStruct((N_NODES, EMB), jnp.float32),
        mesh=mesh,
        scratch_types=[
            [pltpu.VMEM((G, _IW), jnp.int32)] * 2,
            [pltpu.VMEM((G, _IW), jnp.int32)] * 2,
            [pltpu.VMEM((chunk_e, 16), jnp.float32)] * 2,
            [pltpu.VMEM((chunk_e, 16), jnp.float32)] * 2,
            pltpu.VMEM((zrows, 16), jnp.float32),
            pltpu.VMEM_SHARED((N_NODES, 16), jnp.float32),
            [pltpu.SemaphoreType.DMA] * 2,
            [pltpu.SemaphoreType.DMA] * 2,
        ],
        compiler_params=_SC_PARAMS,
    )
    def k(e2_hbm, d_hbm, m_hbm, ie, io, Ae, Ao, zb, table, ld, sc):
        c = lax.axis_index("c")
        s = lax.axis_index("s")

        def zrow(r, carry):
            zb[r, pl.ds(0, 16)] = jnp.zeros((16,), jnp.float32)
            return carry

        lax.fori_loop(0, zrows, zrow, 0)

        for qi in range(2):
            q = c * 2 + qi
            for z in range(stripe // zrows):
                pltpu.sync_copy(zb, table.at[pl.ds(s * stripe + z * zrows,
                                                   zrows)])
            plsc.subcore_barrier()

            def fire_loads(ci, p, q=q):
                r0 = s * rows_per_t + ci * G
                p0 = r0 * _IW
                pltpu.async_copy(d_hbm.at[pl.ds(r0, G)], ie[p], ld[p])
                pltpu.async_copy(d_hbm.at[pl.ds(_HROWS + r0, G)], io[p],
                                 ld[p])
                pltpu.async_copy(
                    e2_hbm.at[pl.ds(p0, pairs), pl.ds(q * 16, 16)],
                    Ae[p], ld[p])
                pltpu.async_copy(
                    e2_hbm.at[pl.ds(p0, pairs), pl.ds(EMB + q * 16, 16)],
                    Ao[p], ld[p])

            def drain_loads(p, q=q):
                pltpu.make_async_copy(d_hbm.at[pl.ds(0, G)], ie[p],
                                      ld[p]).wait()
                pltpu.make_async_copy(d_hbm.at[pl.ds(0, G)], io[p],
                                      ld[p]).wait()
                pltpu.make_async_copy(
                    e2_hbm.at[pl.ds(0, pairs), pl.ds(q * 16, 16)],
                    Ae[p], ld[p]).wait()
                pltpu.make_async_copy(
                    e2_hbm.at[pl.ds(0, pairs), pl.ds(EMB + q * 16, 16)],
                    Ao[p], ld[p]).wait()

            def fire_scatters(p):
                for j in range(G):
                    pltpu.async_copy(Ae[p].at[pl.ds(j * _IW, _IW)],
                                     table.at[ie[p].at[j]], sc[p], add=True)
                    pltpu.async_copy(Ao[p].at[pl.ds(j * _IW, _IW)],
                                     table.at[io[p].at[j]], sc[p], add=True)

            def drain_scatters(p):
                for j in range(G):
                    pltpu.make_async_copy(Ae[p].at[pl.ds(j * _IW, _IW)],
                                          table.at[ie[p].at[j]],
                                          sc[p]).wait()
                    pltpu.make_async_copy(Ao[p].at[pl.ds(j * _IW, _IW)],
                                          table.at[io[p].at[j]],
                                          sc[p]).wait()

            def chunk(ci, carry):
                fire_loads(ci, 0)
                drain_loads(0)
                fire_scatters(0)
                drain_scatters(0)
                return carry

            lax.fori_loop(0, n_chunks, chunk, 0)
            plsc.subcore_barrier()
            pltpu.sync_copy(table.at[pl.ds(s * stripe, stripe)],
                            m_hbm.at[pl.ds(s * stripe, stripe),
                                     pl.ds(q * 16, 16)])
            plsc.subcore_barrier()

    return k(e2, d2d)


def _sc_pool(x1, x2, batch2d):
    G = 5
    chunk_n = G * _IW
    rows_per_t = _NROWS // _NS
    n_chunks = rows_per_t // G
    stripe = N_GRAPHS // _NS
    mesh = plsc.VectorSubcoreMesh(core_axis_name="c", subcore_axis_name="s")

    @functools.partial(
        pl.kernel,
        out_type=[jax.ShapeDtypeStruct((N_GRAPHS, EMB), jnp.float32)] * 2,
        mesh=mesh,
        scratch_types=[
            pltpu.VMEM((G, _IW), jnp.int32),
            pltpu.VMEM((chunk_n, EMB), jnp.float32),
            pltpu.VMEM((stripe, EMB), jnp.float32),
            pltpu.VMEM_SHARED((N_GRAPHS, EMB), jnp.float32),
        ],
        compiler_params=_SC_PARAMS,
    )
    def k(x1_hbm, x2_hbm, idx_hbm, s1_hbm, s2_hbm, idxb, A, zb, table):
        c = lax.axis_index("c")
        s = lax.axis_index("s")

        def zrow(r, carry):
            for q in range(EMB // 16):
                zb[r, pl.ds(q * 16, 16)] = jnp.zeros((16,), jnp.float32)
            return carry

        lax.fori_loop(0, stripe, zrow, 0)
        pltpu.sync_copy(zb, table.at[pl.ds(s * stripe, stripe)])
        plsc.subcore_barrier()

        def run(src, dst):
            def chunk(ci, carry):
                r0 = s * rows_per_t + ci * G
                n0 = r0 * _IW
                pltpu.sync_copy(idx_hbm.at[pl.ds(r0, G)], idxb)
                pltpu.sync_copy(src.at[pl.ds(n0, chunk_n)], A)
                for j in range(G):
                    pltpu.sync_copy(A.at[pl.ds(j * _IW, _IW)],
                                    table.at[idxb.at[j]], add=True)
                return carry

            lax.fori_loop(0, n_chunks, chunk, 0)
            plsc.subcore_barrier()
            pltpu.sync_copy(table.at[pl.ds(s * stripe, stripe)],
                            dst.at[pl.ds(s * stripe, stripe)])

        @pl.when(c == 0)
        def _():
            run(x1_hbm, s1_hbm)

        @pl.when(c == 1)
        def _():
            run(x2_hbm, s2_hbm)

    return k(x1, x2, batch2d)


def _node_enc_body(x_ref, w1_ref, b1_ref, g1_ref, be1_ref, w2_ref, b2_ref,
                   wu_ref, wv_ref, x0_ref, u0_ref, v0_ref):
    z = _dot(x_ref[...], w1_ref[...]) + b1_ref[...]
    h = _ln_relu(z, g1_ref[...], be1_ref[...])
    x0 = _dot(h, w2_ref[...]) + b2_ref[...]
    x0_ref[...] = x0
    u0_ref[...] = _dot(x0, wu_ref[...]).astype(jnp.bfloat16)
    v0_ref[...] = _dot(x0, wv_ref[...]).astype(jnp.bfloat16)


def _node_enc(x, enc, wu, wv):
    (w1, b1, g1, be1), (w2, b2) = enc
    n_blk = N_NODES // _NODE_BLK
    out = pl.pallas_call(
        _node_enc_body,
        grid=(n_blk,),
        in_specs=[
            pl.BlockSpec((_NODE_BLK, 3), lambda i: (i, 0)),
            _full(w1.shape), _full((1, EMB)), _full((1, EMB)), _full((1, EMB)),
            _full(w2.shape), _full((1, EMB)), _full(wu.shape), _full(wv.shape),
        ],
        out_specs=[pl.BlockSpec((_NODE_BLK, EMB), lambda i: (i, 0))] * 3,
        out_shape=[jax.ShapeDtypeStruct((N_NODES, EMB), jnp.float32),
                   jax.ShapeDtypeStruct((N_NODES, EMB), jnp.bfloat16),
                   jax.ShapeDtypeStruct((N_NODES, EMB), jnp.bfloat16)],
    )(x, w1, b1[None], g1[None], be1[None], w2, b2[None], wu, wv)
    return out


def _edge_l2_body(with_e0, g2_ref, e02_ref, wc_ref, b1_ref, g1_ref, be1_ref,
                  w2_ref, b2_ref, out_ref):
    g2 = g2_ref[...].astype(jnp.float32)
    ze = g2[:, :EMB] + b1_ref[...]
    zo = g2[:, EMB:] + b1_ref[...]
    if with_e0:
        e02 = e02_ref[...]
        wc = wc_ref[...]
        ze = ze + _dot(e02[:, :EMB], wc)
        zo = zo + _dot(e02[:, EMB:], wc)
    he = _ln_relu(ze, g1_ref[...], be1_ref[...])
    ho = _ln_relu(zo, g1_ref[...], be1_ref[...])
    ee = _dot(he, w2_ref[...]) + b2_ref[...]
    eo = _dot(ho, w2_ref[...]) + b2_ref[...]
    out_ref[...] = jnp.concatenate([ee, eo], axis=1)


def _edge_l2(g2, net, e02=None):
    (w1, b1, g1, be1), (w2, b2) = net
    with_e0 = e02 is not None
    wc = w1[2 * EMB:] if with_e0 else jnp.zeros((EMB, EMB), jnp.float32)
    if not with_e0:
        e02 = jnp.zeros((1, 2 * EMB), jnp.float32)
    n_blk = _EPAIRS // _EDGE_BLK2
    eblk = pl.BlockSpec((_EDGE_BLK2, 2 * EMB), lambda i: (i, 0))
    e0blk = eblk if with_e0 else _full((1, 2 * EMB))
    return pl.pallas_call(
        functools.partial(_edge_l2_body, with_e0),
        grid=(n_blk,),
        in_specs=[
            eblk, e0blk,
            _full(wc.shape), _full((1, EMB)), _full((1, EMB)), _full((1, EMB)),
            _full(w2.shape), _full((1, EMB)),
        ],
        out_specs=eblk,
        out_shape=jax.ShapeDtypeStruct((_EPAIRS, 2 * EMB), jnp.float32),
    )(g2, e02, wc, b1[None], g1[None], be1[None], w2, b2[None])


def _node_net_body(with_uv, x_ref, m_ref, wa_ref, wb_ref, b1_ref, g1_ref,
                   be1_ref, w2_ref, b2_ref, wu_ref, wv_ref, x1_ref, u_ref,
                   v_ref):
    z = (_dot(x_ref[...], wa_ref[...]) + _dot(m_ref[...], wb_ref[...])
         + b1_ref[...])
    h = _ln_relu(z, g1_ref[...], be1_ref[...])
    x1 = _dot(h, w2_ref[...]) + b2_ref[...]
    x1_ref[...] = x1
    if with_uv:
        u_ref[...] = _dot(x1, wu_ref[...]).astype(jnp.bfloat16)
        v_ref[...] = _dot(x1, wv_ref[...]).astype(jnp.bfloat16)


def _node_net(x, m, net, wu=None, wv=None):
    (w1, b1, g1, be1), (w2, b2) = net
    wa, wb = w1[:EMB], w1[EMB:]
    with_uv = wu is not None
    if not with_uv:
        wu = jnp.zeros((1, EMB), jnp.float32)
        wv = jnp.zeros((1, EMB), jnp.float32)
    n_blk = N_NODES // _NODE_BLK
    blk = pl.BlockSpec((_NODE_BLK, EMB), lambda i: (i, 0))
    out = pl.pallas_call(
        functools.partial(_node_net_body, with_uv),
        grid=(n_blk,),
        in_specs=[
            blk, blk,
            _full(wa.shape), _full(wb.shape), _full((1, EMB)), _full((1, EMB)),
            _full((1, EMB)), _full(w2.shape), _full((1, EMB)),
            _full(wu.shape), _full(wv.shape),
        ],
        out_specs=[blk] * 3,
        out_shape=[jax.ShapeDtypeStruct((N_NODES, EMB), jnp.float32),
                   jax.ShapeDtypeStruct((N_NODES, EMB), jnp.bfloat16),
                   jax.ShapeDtypeStruct((N_NODES, EMB), jnp.bfloat16)],
    )(x, m, wa, wb, b1[None], g1[None], be1[None], w2, b2[None], wu, wv)
    return out if with_uv else (out[0], None, None)


def _reg_body(batch_ref, s1_ref, s2_ref, wr1_ref, rb1_ref, rg1_ref, rbe1_ref,
              wr2_ref, rb2_ref, rg2_ref, rbe2_ref, wr3_ref, rb3_ref,
              out_ref, cnt_ref):
    i = pl.program_id(0)
    nb = pl.num_programs(0)

    @pl.when(i == 0)
    def _init():
        cnt_ref[...] = jnp.zeros_like(cnt_ref)

    b = batch_ref[0]
    ids = jax.lax.broadcasted_iota(jnp.int32, (1, N_GRAPHS), 1)
    hits = (b[0][:, None] == ids).astype(jnp.float32)
    cnt_ref[...] += jnp.sum(hits, axis=0, keepdims=True)

    @pl.when(i == nb - 1)
    def _final():
        s1 = s1_ref[...]
        s2 = s2_ref[...]
        c = jnp.maximum(cnt_ref[0], 1.0)[:, None]
        w = wr1_ref[...]
        z = (_dot(s1, w[:EMB]) + _dot(s2, w[EMB:2 * EMB])
             + _dot(s1 / c, w[2 * EMB:3 * EMB]) + _dot(s2 / c, w[3 * EMB:])
             + rb1_ref[...])
        h = _ln_relu(z, rg1_ref[...], rbe1_ref[...])
        h = _ln_relu(_dot(h, wr2_ref[...]) + rb2_ref[...], rg2_ref[...],
                     rbe2_ref[...])
        out_ref[...] = _dot(h, wr3_ref[...]) + rb3_ref[...]


def _graph_head(batch, s1, s2, reg):
    (wr1, rb1, rg1, rbe1), (wr2, rb2, rg2, rbe2), (wr3, rb3) = reg
    n_blk = N_NODES // _NODE_BLK
    nbins = wr3.shape[1]
    batch3 = batch.reshape(n_blk, 1, _NODE_BLK)
    out, _ = pl.pallas_call(
        _reg_body,
        grid=(n_blk,),
        in_specs=[
            pl.BlockSpec((1, 1, _NODE_BLK), lambda i: (i, 0, 0)),
            _full(s1.shape), _full(s2.shape),
            _full(wr1.shape), _full((1, EMB)), _full((1, EMB)), _full((1, EMB)),
            _full(wr2.shape), _full((1, EMB)), _full((1, EMB)), _full((1, EMB)),
            _full(wr3.shape), _full((1, nbins)),
        ],
        out_specs=[_full((N_GRAPHS, nbins)), _full((1, N_GRAPHS))],
        out_shape=[jax.ShapeDtypeStruct((N_GRAPHS, nbins), jnp.float32),
                   jax.ShapeDtypeStruct((1, N_GRAPHS), jnp.float32)],
    )(batch3, s1, s2, wr1, rb1[None], rg1[None], rbe1[None], wr2, rb2[None],
      rg2[None], rbe2[None], wr3, rb3[None])
    return out


def kernel(x, batch, edge_index, params):
    start = edge_index[0].astype(jnp.int32)
    end = edge_index[1].astype(jnp.int32)
    batch = batch.astype(jnp.int32)
    s2d = start.reshape(_EROWS, _IW)
    d2d = end.reshape(_EROWS, _IW)
    b2d = batch.reshape(_NROWS, _IW)

    enc = params['edge_encoder']
    we1 = enc[0][0]
    x0, u0, v0 = _node_enc(x, params['node_encoder'], we1[:EMB], we1[EMB:])

    g2_0 = _sc_gather_add(u0, v0, s2d, d2d)
    e2_0 = _edge_l2(g2_0, enc)

    net0 = params['node_networks'][0]
    enet0 = params['edge_networks'][0]
    m0 = _sc_segment_sum(e2_0, d2d)
    wg1 = enet0[0][0]
    x1, u1, v1 = _node_net(x0, m0, net0, wg1[:EMB], wg1[EMB:2 * EMB])

    g2_1 = _sc_gather_add(u1, v1, s2d, d2d)
    e2_1 = _edge_l2(g2_1, enet0, e2_0)

    m1 = _sc_segment_sum(e2_1, d2d)
    x2, _, _ = _node_net(x1, m1, params['node_networks'][1])

    s1, s2 = _sc_pool(x1, x2, b2d)

    return _graph_head(batch, s1, s2, params['regression_network'])


# speedup vs baseline: 1.3712x; 1.3712x over previous
import functools

import jax
import jax.numpy as jnp
from jax import lax
from jax.experimental import pallas as pl
from jax.experimental.pallas import tpu as pltpu
from jax.experimental.pallas import tpu_sc as plsc

N_NODES = 50000
N_EDGES = 800000
N_GRAPHS = 512
EMB = 64

_NODE_BLK = 2000
_EPAIRS = N_EDGES // 2
_EDGE_BLK2 = 3200

_NC = 2
_NS = 16
_NW = _NC * _NS
_IW = 125
_EROWS = N_EDGES // _IW
_HROWS = _EROWS // 2
_NROWS = N_NODES // _IW

_SC_PARAMS = pltpu.CompilerParams(use_tc_tiling_on_sc=False)


def _ln_relu(z, g, b):
    mu = jnp.mean(z, axis=-1, keepdims=True)
    var = jnp.var(z, axis=-1, keepdims=True)
    return jax.nn.relu((z - mu) * jax.lax.rsqrt(var + 1e-5) * g + b)


def _dot(a, w):
    return jax.lax.dot_general(a, w, (((1,), (0,)), ((), ())),
                               preferred_element_type=jnp.float32)


def _full(shape):
    return pl.BlockSpec(shape, lambda i: (0,) * len(shape))


def _sc_gather_add(u, v, s2d, d2d):
    pairs = _IW
    rows_per_w = _HROWS // _NW
    mrows = 20
    n_megas = rows_per_w // mrows
    n_chunks = mrows
    mesh = plsc.VectorSubcoreMesh(core_axis_name="c", subcore_axis_name="s")

    @functools.partial(
        pl.kernel,
        out_type=jax.ShapeDtypeStruct((_EPAIRS, 2 * EMB), jnp.float32),
        mesh=mesh,
        scratch_types=[
            [pltpu.VMEM((mrows, _IW), jnp.int32)] * 4,
            [pltpu.VMEM((_IW, EMB), jnp.float32)] * 8,
            [pltpu.VMEM((pairs, 2 * EMB), jnp.float32)] * 2,
            [pltpu.SemaphoreType.DMA] * 2,
            [pltpu.SemaphoreType.DMA] * 2,
        ],
        compiler_params=_SC_PARAMS,
    )
    def k(u_hbm, v_hbm, s_hbm, d_hbm, g2_hbm, IDX, AB, O, sg, so):
        c = lax.axis_index("c")
        s = lax.axis_index("s")
        wid = s * _NC + c
        row_base = wid * rows_per_w
        si, di, si2, di2 = IDX

        def fire(l, p):
            pltpu.async_copy(u_hbm.at[si.at[l]], AB[4 * p + 0], sg[p])
            pltpu.async_copy(v_hbm.at[di.at[l]], AB[4 * p + 1], sg[p])
            pltpu.async_copy(u_hbm.at[si2.at[l]], AB[4 * p + 2], sg[p])
            pltpu.async_copy(v_hbm.at[di2.at[l]], AB[4 * p + 3], sg[p])

        def drain_gather(p):
            for r in range(4):
                pltpu.make_async_copy(u_hbm.at[si.at[0]], AB[4 * p + r],
                                      sg[p]).wait()

        def mega(mi, carry):
            r0 = row_base + mi * mrows
            pbase = r0 * _IW
            pltpu.sync_copy(s_hbm.at[pl.ds(r0, mrows)], si)
            pltpu.sync_copy(d_hbm.at[pl.ds(r0, mrows)], di)
            pltpu.sync_copy(s_hbm.at[pl.ds(_HROWS + r0, mrows)], si2)
            pltpu.sync_copy(d_hbm.at[pl.ds(_HROWS + r0, mrows)], di2)
            fire(0, 0)

            def pair(l2, carry2):
                for p in range(2):
                    l = 2 * l2 + p

                    @pl.when(l < n_chunks - 1)
                    def _fire_next():
                        fire(l + 1, 1 - p)

                    drain_gather(p)

                    @pl.when(l >= 2)
                    def _wait_out():
                        pltpu.make_async_copy(
                            O[p], g2_hbm.at[pl.ds(pbase + (l - 2) * pairs,
                                                  pairs)], so[p]).wait()

                    def addrow(m, c3, p=p):
                        for q in range(EMB // 16):
                            O[p][m, pl.ds(q * 16, 16)] = (
                                AB[4 * p + 0][m, pl.ds(q * 16, 16)]
                                + AB[4 * p + 1][m, pl.ds(q * 16, 16)])
                            O[p][m, pl.ds(EMB + q * 16, 16)] = (
                                AB[4 * p + 2][m, pl.ds(q * 16, 16)]
                                + AB[4 * p + 3][m, pl.ds(q * 16, 16)])
                        return c3

                    lax.fori_loop(0, _IW, addrow, 0)
                    pltpu.async_copy(
                        O[p], g2_hbm.at[pl.ds(pbase + l * pairs, pairs)],
                        so[p])
                return carry2

            lax.fori_loop(0, n_chunks // 2, pair, 0)
            for p in range(2):
                pltpu.make_async_copy(
                    O[p], g2_hbm.at[pl.ds(pbase + (n_chunks - 2 + p) * pairs,
                                          pairs)], so[p]).wait()
            return carry

        lax.fori_loop(0, n_megas, mega, 0)

    return k(u, v, s2d, d2d)


def _sc_segment_sum(e2, d2d):
    G = 10
    chunk_e = G * _IW
    pairs = chunk_e
    rows_per_t = _HROWS // _NS
    n_chunks = rows_per_t // G
    stripe = N_NODES // _NS
    zrows = 625
    mesh = plsc.VectorSubcoreMesh(core_axis_name="c", subcore_axis_name="s")

    @functools.partial(
        pl.kernel,
        out_type=jax.ShapeDtypeStruct((N_NODES, EMB), jnp.float32),
        mesh=mesh,
        scratch_types=[
            [pltpu.VMEM((G, _IW), jnp.int32)] * 2,
            [pltpu.VMEM((G, _IW), jnp.int32)] * 2,
            [pltpu.VMEM((chunk_e, 16), jnp.float32)] * 2,
            [pltpu.VMEM((chunk_e, 16), jnp.float32)] * 2,
            pltpu.VMEM((zrows, 16), jnp.float32),
            pltpu.VMEM_SHARED((N_NODES, 16), jnp.float32),
            [pltpu.SemaphoreType.DMA] * 2,
            [pltpu.SemaphoreType.DMA] * 2,
        ],
        compiler_params=_SC_PARAMS,
    )
    def k(e2_hbm, d_hbm, m_hbm, ie, io, Ae, Ao, zb, table, ld, sc):
        c = lax.axis_index("c")
        s = lax.axis_index("s")

        def zrow(r, carry):
            zb[r, pl.ds(0, 16)] = jnp.zeros((16,), jnp.float32)
            return carry

        lax.fori_loop(0, zrows, zrow, 0)

        for qi in range(2):
            q = c * 2 + qi
            for z in range(stripe // zrows):
                pltpu.sync_copy(zb, table.at[pl.ds(s * stripe + z * zrows,
                                                   zrows)])
            plsc.subcore_barrier()

            def fire_loads(ci, p, q=q):
                r0 = s * rows_per_t + ci * G
                p0 = r0 * _IW
                pltpu.async_copy(d_hbm.at[pl.ds(r0, G)], ie[p], ld[p])
                pltpu.async_copy(d_hbm.at[pl.ds(_HROWS + r0, G)], io[p],
                                 ld[p])
                pltpu.async_copy(
                    e2_hbm.at[pl.ds(p0, pairs), pl.ds(q * 16, 16)],
                    Ae[p], ld[p])
                pltpu.async_copy(
                    e2_hbm.at[pl.ds(p0, pairs), pl.ds(EMB + q * 16, 16)],
                    Ao[p], ld[p])

            def drain_loads(p, q=q):
                pltpu.make_async_copy(d_hbm.at[pl.ds(0, G)], ie[p],
                                      ld[p]).wait()
                pltpu.make_async_copy(d_hbm.at[pl.ds(0, G)], io[p],
                                      ld[p]).wait()
                pltpu.make_async_copy(
                    e2_hbm.at[pl.ds(0, pairs), pl.ds(q * 16, 16)],
                    Ae[p], ld[p]).wait()
                pltpu.make_async_copy(
                    e2_hbm.at[pl.ds(0, pairs), pl.ds(EMB + q * 16, 16)],
                    Ao[p], ld[p]).wait()

            def fire_scatters(p):
                for j in range(G):
                    pltpu.async_copy(Ae[p].at[pl.ds(j * _IW, _IW)],
                                     table.at[ie[p].at[j]], sc[p], add=True)
                    pltpu.async_copy(Ao[p].at[pl.ds(j * _IW, _IW)],
                                     table.at[io[p].at[j]], sc[p], add=True)

            def drain_scatters(p):
                for j in range(G):
                    pltpu.make_async_copy(Ae[p].at[pl.ds(j * _IW, _IW)],
                                          table.at[ie[p].at[j]],
                                          sc[p]).wait()
                    pltpu.make_async_copy(Ao[p].at[pl.ds(j * _IW, _IW)],
                                          table.at[io[p].at[j]],
                                          sc[p]).wait()

            def chunk(ci, carry):
                fire_loads(ci, 0)
                drain_loads(0)
                fire_scatters(0)
                drain_scatters(0)
                return carry

            lax.fori_loop(0, n_chunks, chunk, 0)
            plsc.subcore_barrier()
            pltpu.sync_copy(table.at[pl.ds(s * stripe, stripe)],
                            m_hbm.at[pl.ds(s * stripe, stripe),
                                     pl.ds(q * 16, 16)])
            plsc.subcore_barrier()

    return k(e2, d2d)


def _sc_pool(x1, x2, batch2d):
    G = 5
    chunk_n = G * _IW
    rows_per_t = _NROWS // _NS
    n_chunks = rows_per_t // G
    stripe = N_GRAPHS // _NS
    mesh = plsc.VectorSubcoreMesh(core_axis_name="c", subcore_axis_name="s")

    @functools.partial(
        pl.kernel,
        out_type=[jax.ShapeDtypeStruct((N_GRAPHS, EMB), jnp.float32)] * 2,
        mesh=mesh,
        scratch_types=[
            pltpu.VMEM((G, _IW), jnp.int32),
            pltpu.VMEM((chunk_n, EMB), jnp.float32),
            pltpu.VMEM((stripe, EMB), jnp.float32),
            pltpu.VMEM_SHARED((N_GRAPHS, EMB), jnp.float32),
        ],
        compiler_params=_SC_PARAMS,
    )
    def k(x1_hbm, x2_hbm, idx_hbm, s1_hbm, s2_hbm, idxb, A, zb, table):
        c = lax.axis_index("c")
        s = lax.axis_index("s")

        def zrow(r, carry):
            for q in range(EMB // 16):
                zb[r, pl.ds(q * 16, 16)] = jnp.zeros((16,), jnp.float32)
            return carry

        lax.fori_loop(0, stripe, zrow, 0)
        pltpu.sync_copy(zb, table.at[pl.ds(s * stripe, stripe)])
        plsc.subcore_barrier()

        def run(src, dst):
            def chunk(ci, carry):
                r0 = s * rows_per_t + ci * G
                n0 = r0 * _IW
                pltpu.sync_copy(idx_hbm.at[pl.ds(r0, G)], idxb)
                pltpu.sync_copy(src.at[pl.ds(n0, chunk_n)], A)
                for j in range(G):
                    pltpu.sync_copy(A.at[pl.ds(j * _IW, _IW)],
                                    table.at[idxb.at[j]], add=True)
                return carry

            lax.fori_loop(0, n_chunks, chunk, 0)
            plsc.subcore_barrier()
            pltpu.sync_copy(table.at[pl.ds(s * stripe, stripe)],
                            dst.at[pl.ds(s * stripe, stripe)])

        @pl.when(c == 0)
        def _():
            run(x1_hbm, s1_hbm)

        @pl.when(c == 1)
        def _():
            run(x2_hbm, s2_hbm)

    return k(x1, x2, batch2d)


def _node_enc_body(x_ref, w1_ref, b1_ref, g1_ref, be1_ref, w2_ref, b2_ref,
                   wu_ref, wv_ref, x0_ref, u0_ref, v0_ref):
    z = _dot(x_ref[...], w1_ref[...]) + b1_ref[...]
    h = _ln_relu(z, g1_ref[...], be1_ref[...])
    x0 = _dot(h, w2_ref[...]) + b2_ref[...]
    x0_ref[...] = x0
    u0_ref[...] = _dot(x0, wu_ref[...])
    v0_ref[...] = _dot(x0, wv_ref[...])


def _node_enc(x, enc, wu, wv):
    (w1, b1, g1, be1), (w2, b2) = enc
    n_blk = N_NODES // _NODE_BLK
    out = pl.pallas_call(
        _node_enc_body,
        grid=(n_blk,),
        in_specs=[
            pl.BlockSpec((_NODE_BLK, 3), lambda i: (i, 0)),
            _full(w1.shape), _full((1, EMB)), _full((1, EMB)), _full((1, EMB)),
            _full(w2.shape), _full((1, EMB)), _full(wu.shape), _full(wv.shape),
        ],
        out_specs=[pl.BlockSpec((_NODE_BLK, EMB), lambda i: (i, 0))] * 3,
        out_shape=[jax.ShapeDtypeStruct((N_NODES, EMB), jnp.float32)] * 3,
    )(x, w1, b1[None], g1[None], be1[None], w2, b2[None], wu, wv)
    return out


def _edge_l2_body(with_e0, g2_ref, e02_ref, wc_ref, m_ref, b1_ref, g1_ref,
                  be1_ref, w2_ref, b2_ref, out_ref):
    z = g2_ref[...] + b1_ref[...]
    if with_e0:
        z = z + _dot(e02_ref[...], wc_ref[...])
    m = m_ref[...]
    mu = _dot(z, m)
    s2 = _dot(z * z, m)
    var = s2 - mu * mu
    h = jax.nn.relu((z - mu) * jax.lax.rsqrt(var + 1e-5) * g1_ref[...]
                    + be1_ref[...])
    out_ref[...] = _dot(h, w2_ref[...]) + b2_ref[...]


def _bdiag(w):
    d = jnp.zeros((2 * EMB, 2 * EMB), jnp.float32)
    return d.at[:EMB, :EMB].set(w).at[EMB:, EMB:].set(w)


def _edge_l2(g2, net, e02=None):
    (w1, b1, g1, be1), (w2, b2) = net
    with_e0 = e02 is not None
    wcd = (_bdiag(w1[2 * EMB:]) if with_e0
           else jnp.zeros((2 * EMB, 2 * EMB), jnp.float32))
    if not with_e0:
        e02 = jnp.zeros((1, 2 * EMB), jnp.float32)
    w2d = _bdiag(w2)
    avg = jnp.kron(jnp.eye(2, dtype=jnp.float32),
                   jnp.full((EMB, EMB), 1.0 / EMB, jnp.float32))
    b1d = jnp.concatenate([b1, b1])[None]
    g1d = jnp.concatenate([g1, g1])[None]
    be1d = jnp.concatenate([be1, be1])[None]
    b2d = jnp.concatenate([b2, b2])[None]
    n_blk = _EPAIRS // _EDGE_BLK2
    eblk = pl.BlockSpec((_EDGE_BLK2, 2 * EMB), lambda i: (i, 0))
    e0blk = eblk if with_e0 else _full((1, 2 * EMB))
    return pl.pallas_call(
        functools.partial(_edge_l2_body, with_e0),
        grid=(n_blk,),
        in_specs=[
            eblk, e0blk,
            _full(wcd.shape), _full(avg.shape), _full((1, 2 * EMB)),
            _full((1, 2 * EMB)), _full((1, 2 * EMB)),
            _full(w2d.shape), _full((1, 2 * EMB)),
        ],
        out_specs=eblk,
        out_shape=jax.ShapeDtypeStruct((_EPAIRS, 2 * EMB), jnp.float32),
    )(g2, e02, wcd, avg, b1d, g1d, be1d, w2d, b2d)


def _node_net_body(with_uv, x_ref, m_ref, wa_ref, wb_ref, b1_ref, g1_ref,
                   be1_ref, w2_ref, b2_ref, wu_ref, wv_ref, x1_ref, u_ref,
                   v_ref):
    z = (_dot(x_ref[...], wa_ref[...]) + _dot(m_ref[...], wb_ref[...])
         + b1_ref[...])
    h = _ln_relu(z, g1_ref[...], be1_ref[...])
    x1 = _dot(h, w2_ref[...]) + b2_ref[...]
    x1_ref[...] = x1
    if with_uv:
        u_ref[...] = _dot(x1, wu_ref[...])
        v_ref[...] = _dot(x1, wv_ref[...])


def _node_net(x, m, net, wu=None, wv=None):
    (w1, b1, g1, be1), (w2, b2) = net
    wa, wb = w1[:EMB], w1[EMB:]
    with_uv = wu is not None
    if not with_uv:
        wu = jnp.zeros((1, EMB), jnp.float32)
        wv = jnp.zeros((1, EMB), jnp.float32)
    n_blk = N_NODES // _NODE_BLK
    blk = pl.BlockSpec((_NODE_BLK, EMB), lambda i: (i, 0))
    out = pl.pallas_call(
        functools.partial(_node_net_body, with_uv),
        grid=(n_blk,),
        in_specs=[
            blk, blk,
            _full(wa.shape), _full(wb.shape), _full((1, EMB)), _full((1, EMB)),
            _full((1, EMB)), _full(w2.shape), _full((1, EMB)),
            _full(wu.shape), _full(wv.shape),
        ],
        out_specs=[blk] * 3,
        out_shape=[jax.ShapeDtypeStruct((N_NODES, EMB), jnp.float32)] * 3,
    )(x, m, wa, wb, b1[None], g1[None], be1[None], w2, b2[None], wu, wv)
    return out if with_uv else (out[0], None, None)


def _reg_body(batch_ref, s1_ref, s2_ref, wr1_ref, rb1_ref, rg1_ref, rbe1_ref,
              wr2_ref, rb2_ref, rg2_ref, rbe2_ref, wr3_ref, rb3_ref,
              out_ref, cnt_ref):
    i = pl.program_id(0)
    nb = pl.num_programs(0)

    @pl.when(i == 0)
    def _init():
        cnt_ref[...] = jnp.zeros_like(cnt_ref)

    b = batch_ref[0]
    ids = jax.lax.broadcasted_iota(jnp.int32, (1, N_GRAPHS), 1)
    hits = (b[0][:, None] == ids).astype(jnp.float32)
    cnt_ref[...] += jnp.sum(hits, axis=0, keepdims=True)

    @pl.when(i == nb - 1)
    def _final():
        s1 = s1_ref[...]
        s2 = s2_ref[...]
        c = jnp.maximum(cnt_ref[0], 1.0)[:, None]
        w = wr1_ref[...]
        z = (_dot(s1, w[:EMB]) + _dot(s2, w[EMB:2 * EMB])
             + _dot(s1 / c, w[2 * EMB:3 * EMB]) + _dot(s2 / c, w[3 * EMB:])
             + rb1_ref[...])
        h = _ln_relu(z, rg1_ref[...], rbe1_ref[...])
        h = _ln_relu(_dot(h, wr2_ref[...]) + rb2_ref[...], rg2_ref[...],
                     rbe2_ref[...])
        out_ref[...] = _dot(h, wr3_ref[...]) + rb3_ref[...]


def _graph_head(batch, s1, s2, reg):
    (wr1, rb1, rg1, rbe1), (wr2, rb2, rg2, rbe2), (wr3, rb3) = reg
    n_blk = N_NODES // _NODE_BLK
    nbins = wr3.shape[1]
    batch3 = batch.reshape(n_blk, 1, _NODE_BLK)
    out, _ = pl.pallas_call(
        _reg_body,
        grid=(n_blk,),
        in_specs=[
            pl.BlockSpec((1, 1, _NODE_BLK), lambda i: (i, 0, 0)),
            _full(s1.shape), _full(s2.shape),
            _full(wr1.shape), _full((1, EMB)), _full((1, EMB)), _full((1, EMB)),
            _full(wr2.shape), _full((1, EMB)), _full((1, EMB)), _full((1, EMB)),
            _full(wr3.shape), _full((1, nbins)),
        ],
        out_specs=[_full((N_GRAPHS, nbins)), _full((1, N_GRAPHS))],
        out_shape=[jax.ShapeDtypeStruct((N_GRAPHS, nbins), jnp.float32),
                   jax.ShapeDtypeStruct((1, N_GRAPHS), jnp.float32)],
    )(batch3, s1, s2, wr1, rb1[None], rg1[None], rbe1[None], wr2, rb2[None],
      rg2[None], rbe2[None], wr3, rb3[None])
    return out


def kernel(x, batch, edge_index, params):
    start = edge_index[0].astype(jnp.int32)
    end = edge_index[1].astype(jnp.int32)
    batch = batch.astype(jnp.int32)
    s2d = start.reshape(_EROWS, _IW)
    d2d = end.reshape(_EROWS, _IW)
    b2d = batch.reshape(_NROWS, _IW)

    enc = params['edge_encoder']
    we1 = enc[0][0]
    x0, u0, v0 = _node_enc(x, params['node_encoder'], we1[:EMB], we1[EMB:])

    g2_0 = _sc_gather_add(u0, v0, s2d, d2d)
    e2_0 = _edge_l2(g2_0, enc)

    net0 = params['node_networks'][0]
    enet0 = params['edge_networks'][0]
    m0 = _sc_segment_sum(e2_0, d2d)
    wg1 = enet0[0][0]
    x1, u1, v1 = _node_net(x0, m0, net0, wg1[:EMB], wg1[EMB:2 * EMB])

    g2_1 = _sc_gather_add(u1, v1, s2d, d2d)
    e2_1 = _edge_l2(g2_1, enet0, e2_0)

    m1 = _sc_segment_sum(e2_1, d2d)
    x2, _, _ = _node_net(x1, m1, params['node_networks'][1])

    s1, s2 = _sc_pool(x1, x2, b2d)

    return _graph_head(batch, s1, s2, params['regression_network'])
